# Initial kernel scaffold; baseline (speedup 1.0000x reference)
import jax
import jax.numpy as jnp
from jax.experimental import pallas as pl


def kernel(node_features, edge_features, edge_index, eW1, eb1, eW2, eb2, eg, ebt, nW1, nb1, nW2, nb2, ng, nbt):
    raise NotImplementedError("write your pallas kernel here")


# speedup vs baseline: 1.7013x; 1.7013x over previous
import functools

import jax
import jax.numpy as jnp
from jax import lax
from jax.experimental import pallas as pl
from jax.experimental.pallas import tpu as pltpu
from jax.experimental.pallas import tpu_sc as plsc

_N = 10000
_E = 160000
_D = 256
_L = 15

_EPAD = 163840
_NC, _NS = 2, 16
_NW = _NC * _NS
_GCH = 40
_ECW = _EPAD // _NW

_CSEG = 256
_NSEG = _EPAD // _CSEG
_WSEG = 264
_RROWS = 10496
_ZROW = 10400

_NPAD = 10240
_NGC = 5
_RQ = 12288
_QGC = 3
_KSEG = 8

_BE = 2048
_BN = 1000

_BF = jnp.bfloat16
_F32 = jnp.float32


def _pq_body(nf, wb, wc, p, q):
    x = nf[...].astype(_BF)
    p[...] = jnp.dot(x, wb[...].astype(_BF), preferred_element_type=_F32)
    q[...] = jnp.dot(x, wc[...].astype(_BF), preferred_element_type=_F32)


def _pq_call(nf, wb, wc):
    bs_n = pl.BlockSpec((_BN, _D), lambda i: (i, 0))
    bs_w = pl.BlockSpec((_D, _D), lambda i: (0, 0))
    return pl.pallas_call(
        _pq_body,
        grid=(_N // _BN,),
        in_specs=[bs_n, bs_w, bs_w],
        out_specs=[bs_n, bs_n],
        out_shape=[jax.ShapeDtypeStruct((_N, _D), _F32),
                   jax.ShapeDtypeStruct((_N, _D), _F32)],
        compiler_params=pltpu.CompilerParams(
            dimension_semantics=("arbitrary",)),
    )(nf, wb, wc)


def _mlp_ln_tail(h, w2, b2, gm, bt, res):
    y = jnp.dot(h.astype(_BF), w2[...].astype(_BF),
                preferred_element_type=_F32) + b2[...]
    mu = jnp.mean(y, axis=1, keepdims=True)
    var = jnp.mean((y - mu) ** 2, axis=1, keepdims=True)
    return (y - mu) * lax.rsqrt(var + 1e-5) * gm[...] + bt[...] + res


def _edge_body(nlo_ref, ef, g1, lr, qr, w1, b1, w2, b2, gm, bt, out):
    c = pl.program_id(0)
    x = ef[...]
    h = jnp.dot(x.astype(_BF), w1[...].astype(_BF),
                preferred_element_type=_F32)
    pieces = []
    for k in range(_KSEG):
        nl = pl.multiple_of(nlo_ref[c * _KSEG + k], 8)
        qwin = qr[pl.ds(nl, _WSEG), :]
        stk = (lax.broadcasted_iota(jnp.int32, (_WSEG, _CSEG), 0)
               == lr[k]).astype(_BF)
        pieces.append(lax.dot_general(
            stk, qwin.astype(_BF), (((0,), (0,)), ((), ())),
            preferred_element_type=_F32))
    qd = jnp.concatenate(pieces, axis=0)
    h = jnp.maximum(h + g1[...] + qd + b1[...], 0.0)
    out[...] = _mlp_ln_tail(h, w2, b2, gm, bt, x)


def _edge_call(ef, g1, lr3, qr, nlo, w1, b1, w2, b2, gm, bt):
    bs_e = pl.BlockSpec((_BE, _D), lambda i, s: (i, 0))
    bs_w = pl.BlockSpec((_D, _D), lambda i, s: (0, 0))
    bs_b = pl.BlockSpec((1, _D), lambda i, s: (0, 0))
    grid_spec = pltpu.PrefetchScalarGridSpec(
        num_scalar_prefetch=1,
        grid=(_EPAD // _BE,),
        in_specs=[bs_e, bs_e,
                  pl.BlockSpec((_KSEG, 1, _CSEG), lambda i, s: (i, 0, 0)),
                  pl.BlockSpec((_RQ, _D), lambda i, s: (0, 0)),
                  bs_w, bs_b, bs_w, bs_b, bs_b, bs_b],
        out_specs=bs_e,
    )
    return pl.pallas_call(
        _edge_body,
        grid_spec=grid_spec,
        out_shape=jax.ShapeDtypeStruct((_EPAD, _D), _F32),
        compiler_params=pltpu.CompilerParams(
            dimension_semantics=("arbitrary",)),
    )(nlo, ef, g1, lr3, qr, w1, b1, w2, b2, gm, bt)


def _agg_body(nlo_ref, ef, gr, out):
    c = pl.program_id(0)

    @pl.when(c == 0)
    def _():
        out[...] = jnp.zeros_like(out)

    nl = pl.multiple_of(nlo_ref[c], 8)
    local = gr[0]
    st = (lax.broadcasted_iota(jnp.int32, (_WSEG, _CSEG), 0)
          == local).astype(_BF)
    part = jnp.dot(st, ef[...].astype(_BF), preferred_element_type=_F32)
    out[pl.ds(nl, _WSEG), :] = out[pl.ds(nl, _WSEG), :] + part


def _agg_call(ef, grank3, nlo):
    grid_spec = pltpu.PrefetchScalarGridSpec(
        num_scalar_prefetch=1,
        grid=(_NSEG,),
        in_specs=[pl.BlockSpec((_CSEG, _D), lambda c, s: (c, 0)),
                  pl.BlockSpec((1, 1, _CSEG), lambda c, s: (c, 0, 0))],
        out_specs=pl.BlockSpec((_RROWS, _D), lambda c, s: (0, 0)),
    )
    return pl.pallas_call(
        _agg_body,
        grid_spec=grid_spec,
        out_shape=jax.ShapeDtypeStruct((_RROWS, _D), _F32),
        compiler_params=pltpu.CompilerParams(
            dimension_semantics=("arbitrary",)),
    )(nlo, ef, grank3)


def _node_body(agg, nf, w1a, w1b, b1, w2, b2, gm, bt, out):
    x = nf[...]
    h = (jnp.dot(agg[...].astype(_BF), w1a[...].astype(_BF),
                 preferred_element_type=_F32)
         + jnp.dot(x.astype(_BF), w1b[...].astype(_BF),
                   preferred_element_type=_F32))
    h = jnp.maximum(h + b1[...], 0.0)
    out[...] = _mlp_ln_tail(h, w2, b2, gm, bt, x)


def _node_call(agg, nf, w1a, w1b, b1, w2, b2, gm, bt):
    bs_n = pl.BlockSpec((_BN, _D), lambda i: (i, 0))
    bs_w = pl.BlockSpec((_D, _D), lambda i: (0, 0))
    bs_b = pl.BlockSpec((1, _D), lambda i: (0, 0))
    return pl.pallas_call(
        _node_body,
        grid=(_N // _BN,),
        in_specs=[bs_n, bs_n, bs_w, bs_w, bs_b, bs_w, bs_b, bs_b, bs_b],
        out_specs=bs_n,
        out_shape=jax.ShapeDtypeStruct((_N, _D), _F32),
        compiler_params=pltpu.CompilerParams(
            dimension_semantics=("arbitrary",)),
    )(agg, nf, w1a, w1b, b1, w2, b2, gm, bt)


def _gather_call(p, src3):
    mesh = plsc.VectorSubcoreMesh(core_axis_name="c", subcore_axis_name="s")

    @functools.partial(
        pl.kernel,
        mesh=mesh,
        out_type=jax.ShapeDtypeStruct((_EPAD, _D), _F32),
        scratch_types=[
            pltpu.VMEM((_GCH, 128), jnp.int32),
            pltpu.VMEM((128, _D), _F32),
            pltpu.SemaphoreType.DMA,
        ],
    )
    def k(p_hbm, src_hbm, g_hbm, isrc, rows, sem):
        wid = lax.axis_index("s") * _NC + lax.axis_index("c")
        pltpu.sync_copy(src_hbm.at[wid], isrc)

        def body(j, carry):
            pltpu.async_copy(p_hbm.at[isrc.at[j]], rows, sem).wait()
            out0 = pl.multiple_of(wid * _ECW + j * 128, 128)
            pltpu.sync_copy(rows, g_hbm.at[pl.ds(out0, 128)])
            return carry

        lax.fori_loop(0, _GCH, body, 0)

    return k(p, src3)


def _qrank_call(q, nd3):
    mesh = plsc.VectorSubcoreMesh(core_axis_name="c", subcore_axis_name="s")

    @functools.partial(
        pl.kernel,
        mesh=mesh,
        out_type=jax.ShapeDtypeStruct((_RQ, _D), _F32),
        scratch_types=[
            pltpu.VMEM((_QGC, 128), jnp.int32),
            pltpu.VMEM((128, _D), _F32),
            pltpu.SemaphoreType.DMA,
        ],
    )
    def k(q_hbm, nd_hbm, o_hbm, idx, rows, sem):
        wid = lax.axis_index("s") * _NC + lax.axis_index("c")
        pltpu.sync_copy(nd_hbm.at[wid], idx)

        def body(j, carry):
            pltpu.async_copy(q_hbm.at[idx.at[j]], rows, sem).wait()
            out0 = pl.multiple_of(wid * _QGC * 128 + j * 128, 128)
            pltpu.sync_copy(rows, o_hbm.at[pl.ds(out0, 128)])
            return carry

        lax.fori_loop(0, _QGC, body, 0)

    return k(q, nd3)


def _ngather_call(table, rkn3):
    mesh = plsc.VectorSubcoreMesh(core_axis_name="c", subcore_axis_name="s")

    @functools.partial(
        pl.kernel,
        mesh=mesh,
        out_type=jax.ShapeDtypeStruct((_NPAD, _D), _F32),
        scratch_types=[
            pltpu.VMEM((_NGC, 64), jnp.int32),
            pltpu.VMEM((64, _D), _F32),
            pltpu.SemaphoreType.DMA,
        ],
    )
    def k(t_hbm, r_hbm, o_hbm, idx, rows, sem):
        wid = lax.axis_index("s") * _NC + lax.axis_index("c")
        pltpu.sync_copy(r_hbm.at[wid], idx)

        def body(j, carry):
            pltpu.async_copy(t_hbm.at[idx.at[j]], rows, sem).wait()
            out0 = pl.multiple_of(wid * _NGC * 64 + j * 64, 64)
            pltpu.sync_copy(rows, o_hbm.at[pl.ds(out0, 64)])
            return carry

        lax.fori_loop(0, _NGC, body, 0)

    return k(table, rkn3)


def kernel(node_features, edge_features, edge_index, eW1, eb1, eW2, eb2,
           eg, ebt, nW1, nb1, nW2, nb2, ng, nbt):
    src = edge_index[0]
    dst = edge_index[1]

    pad = _EPAD - _E
    dstp = jnp.concatenate([dst, jnp.full((pad,), _N, jnp.int32)])
    srcp = jnp.pad(src, (0, pad))
    order = jnp.argsort(dstp)
    dst_s = dstp[order]
    src_s = srcp[order]
    ef = jnp.pad(edge_features, ((0, pad), (0, 0)))[order]

    gsrc3 = src_s.reshape(_NW, _GCH, 128)

    grank = jnp.concatenate(
        [jnp.zeros((1,), jnp.int32),
         jnp.cumsum((dst_s[1:] != dst_s[:-1]).astype(jnp.int32))])
    nlo = (grank[::_CSEG] // 8) * 8
    local = grank - jnp.repeat(nlo, _CSEG)
    grank3 = local.reshape(_NSEG, 1, _CSEG)

    nn = jnp.arange(_N, dtype=jnp.int32)
    e0 = jnp.minimum(jnp.searchsorted(dst_s, nn), _EPAD - 1)
    present = dst_s[e0] == nn
    rkn = jnp.where(present, grank[e0], _ZROW)
    rkn3 = jnp.pad(rkn, (0, _NPAD - _N),
                   constant_values=_ZROW).reshape(_NW, _NGC, 64)

    e_r = jnp.searchsorted(grank, jnp.arange(_RQ, dtype=jnp.int32))
    nd = dst_s[jnp.minimum(e_r, _EPAD - 1)]
    nd3 = jnp.where(nd >= _N, 0, nd).reshape(_NW, _QGC, 128)

    nf = node_features

    eW1a = eW1[:, :_D]
    eW1b = eW1[:, _D:2 * _D]
    eW1c = eW1[:, 2 * _D:]
    nW1a = nW1[:, :_D]
    nW1b = nW1[:, _D:]
    eb1r, eb2r, egr, ebtr = (v.reshape(_L, 1, _D) for v in (eb1, eb2, eg, ebt))
    nb1r, nb2r, ngr, nbtr = (v.reshape(_L, 1, _D) for v in (nb1, nb2, ng, nbt))

    for i in range(_L):
        p, q = _pq_call(nf, eW1b[i], eW1c[i])
        g1 = _gather_call(p, gsrc3)
        qrk = _qrank_call(q, nd3)
        ef = _edge_call(ef, g1, grank3, qrk, nlo, eW1a[i], eb1r[i], eW2[i],
                        eb2r[i], egr[i], ebtr[i])
        ranked = _agg_call(ef, grank3, nlo)
        agg = _ngather_call(ranked, rkn3)
        nf = _node_call(agg[:_N], nf, nW1a[i], nW1b[i], nb1r[i], nW2[i],
                        nb2r[i], ngr[i], nbtr[i])
    return nf
